# baseline (speedup 1.0000x reference)
import functools

import jax
import jax.numpy as jnp
from jax import lax
from jax.experimental import pallas as pl
from jax.experimental.pallas import tpu as pltpu
from jax.experimental.pallas import tpu_sc as plsc


K = 16
NBUF = 3


def _make_gather(vocab: int, d_model: int, n_ids: int):
  info = plsc.get_sparse_core_info()
  nw = info.num_cores * info.num_subcores
  b_per_w = n_ids // nw
  nch = b_per_w // K

  mesh = plsc.VectorSubcoreMesh(core_axis_name="c", subcore_axis_name="s")

  @functools.partial(
      pl.kernel,
      out_type=jax.ShapeDtypeStruct((n_ids, d_model), jnp.float32),
      mesh=mesh,
      scratch_types=[
          pltpu.VMEM((nch, K), jnp.int32),
          *[pltpu.VMEM((K, d_model), jnp.float32) for _ in range(NBUF)],
          *[pltpu.SemaphoreType.DMA for _ in range(2 * NBUF)],
      ],
  )
  def gather_kernel(ids_hbm, table_hbm, out_hbm, idx_v, *rest):
    bufs = rest[:NBUF]
    in_sems = rest[NBUF:2 * NBUF]
    out_sems = rest[2 * NBUF:]
    wid = lax.axis_index("s") * info.num_cores + lax.axis_index("c")
    base = wid * b_per_w

    def gather(c, b):
      pltpu.async_copy(table_hbm.at[idx_v.at[c]], bufs[b], in_sems[b])

    def gather_wait(c, b):
      pltpu.make_async_copy(table_hbm.at[idx_v.at[c]], bufs[b],
                            in_sems[b]).wait()

    def put(c, b):
      pltpu.async_copy(bufs[b], out_hbm.at[pl.ds(base + c * K, K)],
                       out_sems[b])

    def put_wait(c, b):
      pltpu.make_async_copy(bufs[b], out_hbm.at[pl.ds(base + c * K, K)],
                            out_sems[b]).wait()

    pltpu.sync_copy(ids_hbm.at[wid], idx_v)

    for b in range(NBUF - 1):
      gather(b, b)

    nmain = (nch // NBUF) * NBUF

    @pl.loop(0, nmain, step=NBUF)
    def _(c0):
      for b in range(NBUF):
        c = c0 + b
        gather_wait(c, b)
        put(c, b)

        @pl.when(c > 0)
        def _():
          put_wait(c - 1, (b - 1) % NBUF)

        @pl.when(c + NBUF - 1 < nch)
        def _():
          gather(c + NBUF - 1, (b - 1) % NBUF)

    for c in range(nmain, nch):
      b = c % NBUF
      gather_wait(c, b)
      put(c, b)
      put_wait(c - 1, (c - 1) % NBUF)

    put_wait(nch - 1, (nch - 1) % NBUF)

  return gather_kernel


R_TC = 8


def _make_tc_gather(vocab: int, n_rows: int, d_model: int):
  nsteps = n_rows // R_TC

  def body(idx_ref, table_ref, out_ref, sem):
    i = pl.program_id(0)
    copies = [
        pltpu.make_async_copy(
            table_ref.at[pl.ds(idx_ref[i * R_TC + j], 1)],
            out_ref.at[pl.ds(j, 1)], sem)
        for j in range(R_TC)
    ]
    for c in copies:
      c.start()
    for c in copies:
      c.wait()

  return pl.pallas_call(
      body,
      grid_spec=pltpu.PrefetchScalarGridSpec(
          num_scalar_prefetch=1,
          grid=(nsteps,),
          in_specs=[pl.BlockSpec(memory_space=pl.ANY)],
          out_specs=pl.BlockSpec((R_TC, d_model), lambda i, idx: (i, 0)),
          scratch_shapes=[pltpu.SemaphoreType.DMA],
      ),
      out_shape=jax.ShapeDtypeStruct((n_rows, d_model), jnp.float32),
  )


def _make_rowdma_gather(vocab: int, d_model: int, n_ids: int):
  info = plsc.get_sparse_core_info()
  nw = info.num_cores * info.num_subcores
  b_per_w = n_ids // nw
  BATCH = 16
  mesh = plsc.VectorSubcoreMesh(core_axis_name="c", subcore_axis_name="s")

  @functools.partial(
      pl.kernel,
      out_type=jax.ShapeDtypeStruct((n_ids * d_model,), jnp.float32),
      mesh=mesh,
      scratch_types=[
          pltpu.VMEM((b_per_w,), jnp.int32),
          pltpu.SemaphoreType.DMA,
      ],
  )
  def gather_kernel(ids_hbm, table_hbm, out_hbm, idx_v, sem):
    wid = lax.axis_index("s") * info.num_cores + lax.axis_index("c")
    base = wid * b_per_w
    pltpu.sync_copy(ids_hbm.at[pl.ds(base, b_per_w)], idx_v)

    @pl.loop(0, b_per_w, step=BATCH)
    def _(r0):
      vec = idx_v[pl.ds(r0, BATCH)]
      copies = []
      for j in range(BATCH):
        copies.append(pltpu.make_async_copy(
            table_hbm.at[pl.ds(vec[j] * d_model, d_model)],
            out_hbm.at[pl.ds((base + r0 + j) * d_model, d_model)], sem))
      for c in copies:
        c.start()
      for c in copies:
        c.wait()

  return gather_kernel


def kernel(input_ids, table):
  vocab, d_model = table.shape
  n_ids = input_ids.size
  ids = input_ids.reshape(-1).astype(jnp.int32)
  out = _make_rowdma_gather(vocab, d_model, n_ids)(
      ids, table.reshape(-1))
  return out.reshape(*input_ids.shape, d_model)


# speedup vs baseline: 42.3462x; 42.3462x over previous
import functools

import jax
import jax.numpy as jnp
from jax import lax
from jax.experimental import pallas as pl
from jax.experimental.pallas import tpu as pltpu
from jax.experimental.pallas import tpu_sc as plsc

K = 16
NBUF = 3


def _make_gather(vocab: int, d_model: int, n_ids: int):
  info = plsc.get_sparse_core_info()
  nw = info.num_cores * info.num_subcores
  b_per_w = n_ids // nw
  nch = b_per_w // K

  mesh = plsc.VectorSubcoreMesh(core_axis_name="c", subcore_axis_name="s")

  @functools.partial(
      pl.kernel,
      out_type=jax.ShapeDtypeStruct((n_ids, d_model), jnp.float32),
      mesh=mesh,
      scratch_types=[
          pltpu.VMEM((nch, K), jnp.int32),
          *[pltpu.VMEM((K, d_model), jnp.float32) for _ in range(NBUF)],
          *[pltpu.SemaphoreType.DMA for _ in range(2 * NBUF)],
      ],
  )
  def gather_kernel(ids_hbm, table_hbm, out_hbm, idx_v, *rest):
    bufs = rest[:NBUF]
    in_sems = rest[NBUF:2 * NBUF]
    out_sems = rest[2 * NBUF:]
    wid = lax.axis_index("s") * info.num_cores + lax.axis_index("c")
    base = wid * b_per_w

    def gather(c, b):
      pltpu.async_copy(table_hbm.at[idx_v.at[c]], bufs[b], in_sems[b])

    def gather_wait(c, b):
      pltpu.make_async_copy(table_hbm.at[idx_v.at[c]], bufs[b],
                            in_sems[b]).wait()

    def put(c, b):
      pltpu.async_copy(bufs[b], out_hbm.at[pl.ds(base + c * K, K)],
                       out_sems[b])

    def put_wait(c, b):
      pltpu.make_async_copy(bufs[b], out_hbm.at[pl.ds(base + c * K, K)],
                            out_sems[b]).wait()

    pltpu.sync_copy(ids_hbm.at[wid], idx_v)

    for b in range(NBUF - 1):
      gather(b, b)

    nmain = (nch // NBUF) * NBUF

    @pl.loop(0, nmain, step=NBUF)
    def _(c0):
      for b in range(NBUF):
        c = c0 + b
        gather_wait(c, b)
        put(c, b)

        @pl.when(c > 0)
        def _():
          put_wait(c - 1, (b - 1) % NBUF)

        @pl.when(c + NBUF - 1 < nch)
        def _():
          gather(c + NBUF - 1, (b - 1) % NBUF)

    for c in range(nmain, nch):
      b = c % NBUF
      gather_wait(c, b)
      put(c, b)
      put_wait(c - 1, (c - 1) % NBUF)

    put_wait(nch - 1, (nch - 1) % NBUF)

  return gather_kernel


def kernel(input_ids, table):
  vocab, d_model = table.shape
  n_ids = input_ids.size
  info = plsc.get_sparse_core_info()
  nw = info.num_cores * info.num_subcores
  nch = n_ids // (nw * K)
  ids3 = input_ids.reshape(nw, nch, K).astype(jnp.int32)
  out = _make_gather(vocab, d_model, n_ids)(ids3, table)
  return out.reshape(*input_ids.shape, d_model)


# speedup vs baseline: 42.5578x; 1.0050x over previous
import functools

import jax
import jax.numpy as jnp
from jax import lax
from jax.experimental import pallas as pl
from jax.experimental.pallas import tpu as pltpu
from jax.experimental.pallas import tpu_sc as plsc

K = 16
NBUF = 3


def _make_gather(vocab: int, d_model: int, n_ids: int):
  info = plsc.get_sparse_core_info()
  nw = info.num_cores * info.num_subcores
  b_per_w = n_ids // nw
  nch = b_per_w // K

  mesh = plsc.VectorSubcoreMesh(core_axis_name="c", subcore_axis_name="s")

  @functools.partial(
      pl.kernel,
      out_type=jax.ShapeDtypeStruct((n_ids, d_model), jnp.float32),
      mesh=mesh,
      scratch_types=[
          pltpu.VMEM((b_per_w,), jnp.int32),
          *[pltpu.VMEM((K, d_model), jnp.float32) for _ in range(NBUF)],
          *[pltpu.SemaphoreType.DMA for _ in range(2 * NBUF)],
      ],
  )
  def gather_kernel(ids_hbm, table_hbm, out_hbm, idx_v, *rest):
    bufs = rest[:NBUF]
    in_sems = rest[NBUF:2 * NBUF]
    out_sems = rest[2 * NBUF:]
    wid = lax.axis_index("s") * info.num_cores + lax.axis_index("c")
    base = wid * b_per_w

    def gather(c, b):
      pltpu.async_copy(table_hbm.at[idx_v.at[pl.ds(c * K, K)]], bufs[b],
                       in_sems[b])

    def gather_wait(c, b):
      pltpu.make_async_copy(table_hbm.at[idx_v.at[pl.ds(c * K, K)]], bufs[b],
                            in_sems[b]).wait()

    def put(c, b):
      pltpu.async_copy(bufs[b], out_hbm.at[pl.ds(base + c * K, K)],
                       out_sems[b])

    def put_wait(c, b):
      pltpu.make_async_copy(bufs[b], out_hbm.at[pl.ds(base + c * K, K)],
                            out_sems[b]).wait()

    pltpu.sync_copy(ids_hbm.at[pl.ds(base, b_per_w)], idx_v)

    for b in range(NBUF - 1):
      gather(b, b)

    nmain = (nch // NBUF) * NBUF

    @pl.loop(0, nmain, step=NBUF)
    def _(c0):
      for b in range(NBUF):
        c = c0 + b
        gather_wait(c, b)
        put(c, b)

        @pl.when(c > 0)
        def _():
          put_wait(c - 1, (b - 1) % NBUF)

        @pl.when(c + NBUF - 1 < nch)
        def _():
          gather(c + NBUF - 1, (b - 1) % NBUF)

    for c in range(nmain, nch):
      b = c % NBUF
      gather_wait(c, b)
      put(c, b)
      put_wait(c - 1, (c - 1) % NBUF)

    put_wait(nch - 1, (nch - 1) % NBUF)

  return gather_kernel


def kernel(input_ids, table):
  vocab, d_model = table.shape
  n_ids = input_ids.size
  info = plsc.get_sparse_core_info()
  nw = info.num_cores * info.num_subcores
  ids = input_ids.reshape(-1).astype(jnp.int32)
  out = _make_gather(vocab, d_model, n_ids)(ids, table)
  return out.reshape(*input_ids.shape, d_model)
